# baseline (speedup 1.0000x reference)
import jax
import jax.numpy as jnp
from jax import lax
from jax.experimental import pallas as pl
from jax.experimental.pallas import tpu as pltpu
from jax.experimental.pallas import tpu_sc as plsc

_B = 16384
_V = 256
_NC = 2
_NS = 16
_NW = _NC * _NS
_BPW = _B // _NW
_L = 16
_CHUNK = 128
_NCHUNK = _BPW // _CHUNK


def _gather_body(x0_hbm, x1_hbm, x2_hbm, board_hbm, out_hbm,
                 x0_v, x1_v, x2_v, idx_v, val_v, sem):
    wid = lax.axis_index("s") * _NC + lax.axis_index("c")
    base = wid * _BPW

    pltpu.sync_copy(x0_hbm.at[pl.ds(base, _BPW)], x0_v)
    pltpu.sync_copy(x1_hbm.at[pl.ds(base, _BPW)], x1_v)
    pltpu.sync_copy(x2_hbm.at[pl.ds(base, _BPW)], x2_v)

    for i in range(_BPW // _L):
        s = pl.ds(i * _L, _L)
        idx_v[s] = (x0_v[s] << 16) | (x1_v[s] << 8) | x2_v[s]

    copies = []
    for j in range(_NCHUNK):
        c = pl.ds(j * _CHUNK, _CHUNK)
        copies.append(
            pltpu.async_copy(board_hbm.at[idx_v.at[c]], val_v.at[c], sem))
    for cp in copies:
        cp.wait()

    pltpu.sync_copy(val_v, out_hbm.at[pl.ds(base, _BPW)])


@jax.jit
def _gather_sc(x0, x1, x2, board_flat):
    mesh = plsc.VectorSubcoreMesh(core_axis_name="c", subcore_axis_name="s")
    f = pl.kernel(
        _gather_body,
        out_type=jax.ShapeDtypeStruct((_B,), jnp.float32),
        mesh=mesh,
        scratch_types=[
            pltpu.VMEM((_BPW,), jnp.int32),
            pltpu.VMEM((_BPW,), jnp.int32),
            pltpu.VMEM((_BPW,), jnp.int32),
            pltpu.VMEM((_BPW,), jnp.int32),
            pltpu.VMEM((_BPW,), jnp.float32),
            pltpu.SemaphoreType.DMA,
        ],
    )
    return f(x0, x1, x2, board_flat)


def kernel(x0, x1, x2, board):
    x0 = x0.astype(jnp.int32)
    x1 = x1.astype(jnp.int32)
    x2 = x2.astype(jnp.int32)
    board_flat = board.reshape(_V * _V * _V)
    out = _gather_sc(x0, x1, x2, board_flat)
    return out[:, None]


# speedup vs baseline: 2.4563x; 2.4563x over previous
import jax
import jax.numpy as jnp
from jax import lax
from jax.experimental import pallas as pl
from jax.experimental.pallas import tpu as pltpu
from jax.experimental.pallas import tpu_sc as plsc

_B = 16384
_V = 256
_NC = 2
_NS = 16
_NW = _NC * _NS
_BPW = _B // _NW
_L = 16
_NROWS = _V * _V
_CHUNK = 128
_NCHUNK = _BPW // _CHUNK


def _gather_body(x0_hbm, x1_hbm, x2_hbm, board_hbm, out_hbm,
                 x0_v, x1_v, x2_v, row_v, col_v, rows_a, rows_b, val_v,
                 sem_a, sem_b):
    wid = lax.axis_index("s") * _NC + lax.axis_index("c")
    base = wid * _BPW
    board_rows = board_hbm.reshape(_NROWS, _V)

    pltpu.sync_copy(x0_hbm.at[pl.ds(base, _BPW)], x0_v)
    pltpu.sync_copy(x1_hbm.at[pl.ds(base, _BPW)], x1_v)
    pltpu.sync_copy(x2_hbm.at[pl.ds(base, _BPW)], x2_v)

    for i in range(_BPW // _L):
        s = pl.ds(i * _L, _L)
        row_v[s] = (x0_v[s] << 8) | x1_v[s]
        col_v[s] = x2_v[s]

    bufs = (rows_a, rows_b)
    sems = (sem_a, sem_b)

    def fire(j):
        c = pl.ds(j * _CHUNK, _CHUNK)
        return pltpu.async_copy(board_rows.at[row_v.at[c]],
                                bufs[j % 2],
                                sems[j % 2])

    lane = lax.iota(jnp.int32, _L)
    copies = [fire(0), fire(1)]
    for j in range(_NCHUNK):
        copies[j % 2].wait()
        for i in range(_CHUNK // _L):
            g = j * _CHUNK + i * _L
            s = pl.ds(g, _L)
            val_v[s] = plsc.load_gather(bufs[j % 2],
                                        [lane + i * _L, col_v[s]])
        if j + 2 < _NCHUNK:
            copies[j % 2] = fire(j + 2)

    pltpu.sync_copy(val_v, out_hbm.at[pl.ds(base, _BPW)])


@jax.jit
def _gather_sc(x0, x1, x2, board):
    mesh = plsc.VectorSubcoreMesh(core_axis_name="c", subcore_axis_name="s")
    f = pl.kernel(
        _gather_body,
        out_type=jax.ShapeDtypeStruct((_B,), jnp.float32),
        mesh=mesh,
        compiler_params=pltpu.CompilerParams(needs_layout_passes=False),
        scratch_types=[
            pltpu.VMEM((_BPW,), jnp.int32),
            pltpu.VMEM((_BPW,), jnp.int32),
            pltpu.VMEM((_BPW,), jnp.int32),
            pltpu.VMEM((_BPW,), jnp.int32),
            pltpu.VMEM((_BPW,), jnp.int32),
            pltpu.VMEM((_CHUNK, _V), jnp.float32),
            pltpu.VMEM((_CHUNK, _V), jnp.float32),
            pltpu.VMEM((_BPW,), jnp.float32),
            pltpu.SemaphoreType.DMA,
            pltpu.SemaphoreType.DMA,
        ],
    )
    return f(x0, x1, x2, board)


def kernel(x0, x1, x2, board):
    x0 = x0.astype(jnp.int32)
    x1 = x1.astype(jnp.int32)
    x2 = x2.astype(jnp.int32)
    out = _gather_sc(x0, x1, x2, board)
    return out[:, None]


# speedup vs baseline: 2.5265x; 1.0286x over previous
import jax
import jax.numpy as jnp
from jax import lax
from jax.experimental import pallas as pl
from jax.experimental.pallas import tpu as pltpu
from jax.experimental.pallas import tpu_sc as plsc

_B = 16384
_V = 256
_NC = 2
_NS = 16
_NW = _NC * _NS
_BPW = _B // _NW
_L = 16
_NROWS = _V * _V
_CHUNK = 64
_NCHUNK = _BPW // _CHUNK
_NBUF = 4


def _gather_body(x0_hbm, x1_hbm, x2_hbm, board_hbm, out_hbm,
                 x0_v, x1_v, x2_v, row_v, val_v,
                 rows_0, rows_1, rows_2, rows_3,
                 sem_in, sem_0, sem_1, sem_2, sem_3):
    wid = lax.axis_index("s") * _NC + lax.axis_index("c")
    base = wid * _BPW
    board_rows = board_hbm.reshape(_NROWS, _V)

    c0 = pltpu.async_copy(x0_hbm.at[pl.ds(base, _BPW)], x0_v, sem_in)
    c1 = pltpu.async_copy(x1_hbm.at[pl.ds(base, _BPW)], x1_v, sem_in)
    c2 = pltpu.async_copy(x2_hbm.at[pl.ds(base, _BPW)], x2_v, sem_in)
    c0.wait()
    c1.wait()

    for i in range(_BPW // _L):
        s = pl.ds(i * _L, _L)
        row_v[s] = (x0_v[s] << 8) | x1_v[s]

    bufs = (rows_0, rows_1, rows_2, rows_3)
    sems = (sem_0, sem_1, sem_2, sem_3)

    def fire(j):
        c = pl.ds(j * _CHUNK, _CHUNK)
        return pltpu.async_copy(board_rows.at[row_v.at[c]],
                                bufs[j % _NBUF], sems[j % _NBUF])

    copies = [fire(j) for j in range(_NBUF)]
    c2.wait()

    lane = lax.iota(jnp.int32, _L)
    for j in range(_NCHUNK):
        copies[j % _NBUF].wait()
        for i in range(_CHUNK // _L):
            g = j * _CHUNK + i * _L
            s = pl.ds(g, _L)
            val_v[s] = plsc.load_gather(bufs[j % _NBUF],
                                        [lane + i * _L, x2_v[s]])
        if j + _NBUF < _NCHUNK:
            copies[j % _NBUF] = fire(j + _NBUF)

    pltpu.sync_copy(val_v, out_hbm.at[pl.ds(base, _BPW)])


@jax.jit
def _gather_sc(x0, x1, x2, board):
    mesh = plsc.VectorSubcoreMesh(core_axis_name="c", subcore_axis_name="s")
    f = pl.kernel(
        _gather_body,
        out_type=jax.ShapeDtypeStruct((_B,), jnp.float32),
        mesh=mesh,
        compiler_params=pltpu.CompilerParams(needs_layout_passes=False),
        scratch_types=[
            pltpu.VMEM((_BPW,), jnp.int32),
            pltpu.VMEM((_BPW,), jnp.int32),
            pltpu.VMEM((_BPW,), jnp.int32),
            pltpu.VMEM((_BPW,), jnp.int32),
            pltpu.VMEM((_BPW,), jnp.float32),
            pltpu.VMEM((_CHUNK, _V), jnp.float32),
            pltpu.VMEM((_CHUNK, _V), jnp.float32),
            pltpu.VMEM((_CHUNK, _V), jnp.float32),
            pltpu.VMEM((_CHUNK, _V), jnp.float32),
            pltpu.SemaphoreType.DMA,
            pltpu.SemaphoreType.DMA,
            pltpu.SemaphoreType.DMA,
            pltpu.SemaphoreType.DMA,
            pltpu.SemaphoreType.DMA,
        ],
    )
    return f(x0, x1, x2, board)


def kernel(x0, x1, x2, board):
    x0 = x0.astype(jnp.int32)
    x1 = x1.astype(jnp.int32)
    x2 = x2.astype(jnp.int32)
    out = _gather_sc(x0, x1, x2, board)
    return out[:, None]


# speedup vs baseline: 2.5733x; 1.0186x over previous
import jax
import jax.numpy as jnp
from jax import lax
from jax.experimental import pallas as pl
from jax.experimental.pallas import tpu as pltpu
from jax.experimental.pallas import tpu_sc as plsc

_B = 16384
_V = 256
_NC = 2
_NS = 16
_NW = _NC * _NS
_BPW = _B // _NW
_L = 16
_NROWS = _V * _V
_CHUNK = 64
_NCHUNK = _BPW // _CHUNK
_NBUF = 5
_GPC = _CHUNK // _L


def _gather_body(x0_hbm, x1_hbm, x2_hbm, board_hbm, out_hbm,
                 x0_v, x2_v, row_v, val_v, bufs, sem_in, sem_out, sems):
    wid = lax.axis_index("s") * _NC + lax.axis_index("c")
    base = wid * _BPW
    board_rows = board_hbm.reshape(_NROWS, _V)

    c0 = pltpu.async_copy(x0_hbm.at[pl.ds(base, _BPW)], x0_v, sem_in)
    c1 = pltpu.async_copy(x1_hbm.at[pl.ds(base, _BPW)], row_v, sem_in)
    c2 = pltpu.async_copy(x2_hbm.at[pl.ds(base, _BPW)], x2_v, sem_out)
    c0.wait()
    c1.wait()

    def compute_rows(j):
        for i in range(_GPC):
            s = pl.ds(j * _CHUNK + i * _L, _L)
            row_v[s] = (x0_v[s] << 8) | row_v[s]

    def fire(j, slot):
        c = pl.ds(j * _CHUNK, _CHUNK)
        return pltpu.async_copy(board_rows.at[row_v.at[c]],
                                bufs.at[slot], sems.at[slot])

    copies = []
    for j in range(_NBUF):
        compute_rows(j)
        copies.append(fire(j, j))
    for j in range(_NBUF, _NCHUNK):
        compute_rows(j)
    c2.wait()

    lane = lax.iota(jnp.int32, _L)
    outs = []
    for j in range(_NCHUNK):
        slot = j % _NBUF
        copies[slot].wait()
        for i in range(_GPC):
            g = j * _CHUNK + i * _L
            s = pl.ds(g, _L)
            val_v[s] = plsc.load_gather(bufs.at[slot],
                                        [lane + i * _L, x2_v[s]])
        if j + _NBUF < _NCHUNK:
            copies[slot] = fire(j + _NBUF, slot)
        c = pl.ds(j * _CHUNK, _CHUNK)
        outs.append(pltpu.async_copy(
            val_v.at[c], out_hbm.at[pl.ds(base + j * _CHUNK, _CHUNK)],
            sem_out))
    for o in outs:
        o.wait()


@jax.jit
def _gather_sc(x0, x1, x2, board):
    mesh = plsc.VectorSubcoreMesh(core_axis_name="c", subcore_axis_name="s")
    f = pl.kernel(
        _gather_body,
        out_type=jax.ShapeDtypeStruct((_B,), jnp.float32),
        mesh=mesh,
        compiler_params=pltpu.CompilerParams(needs_layout_passes=False),
        scratch_types=[
            pltpu.VMEM((_BPW,), jnp.int32),
            pltpu.VMEM((_BPW,), jnp.int32),
            pltpu.VMEM((_BPW,), jnp.int32),
            pltpu.VMEM((_BPW,), jnp.float32),
            pltpu.VMEM((_NBUF, _CHUNK, _V), jnp.float32),
            pltpu.SemaphoreType.DMA,
            pltpu.SemaphoreType.DMA,
            pltpu.SemaphoreType.DMA((_NBUF,)),
        ],
    )
    return f(x0, x1, x2, board)


def kernel(x0, x1, x2, board):
    x0 = x0.astype(jnp.int32)
    x1 = x1.astype(jnp.int32)
    x2 = x2.astype(jnp.int32)
    out = _gather_sc(x0, x1, x2, board)
    return out[:, None]


# speedup vs baseline: 2.5938x; 1.0079x over previous
import jax
import jax.numpy as jnp
from jax import lax
from jax.experimental import pallas as pl
from jax.experimental.pallas import tpu as pltpu
from jax.experimental.pallas import tpu_sc as plsc

_B = 16384
_V = 256
_NC = 2
_NS = 16
_NW = _NC * _NS
_BPW = _B // _NW
_L = 16
_NROWS = _V * _V
_CHUNK = 128
_NCHUNK = _BPW // _CHUNK
_NBUF = 3
_GPC = _CHUNK // _L


def _gather_body(x0_hbm, x1_hbm, x2_hbm, board_hbm, out_hbm,
                 x0_v, x2_v, row_v, val_v, bufs, sem_in, sem_out, sems):
    wid = lax.axis_index("s") * _NC + lax.axis_index("c")
    base = wid * _BPW
    board_rows = board_hbm.reshape(_NROWS, _V)

    c0 = pltpu.async_copy(x0_hbm.at[pl.ds(base, _BPW)], x0_v, sem_in)
    c1 = pltpu.async_copy(x1_hbm.at[pl.ds(base, _BPW)], row_v, sem_in)
    c2 = pltpu.async_copy(x2_hbm.at[pl.ds(base, _BPW)], x2_v, sem_out)
    c0.wait()
    c1.wait()

    def compute_rows(j):
        for i in range(_GPC):
            s = pl.ds(j * _CHUNK + i * _L, _L)
            row_v[s] = (x0_v[s] << 8) | row_v[s]

    def fire(j, slot):
        c = pl.ds(j * _CHUNK, _CHUNK)
        return pltpu.async_copy(board_rows.at[row_v.at[c]],
                                bufs.at[slot], sems.at[slot])

    copies = []
    for j in range(_NBUF):
        compute_rows(j)
        copies.append(fire(j, j))
    for j in range(_NBUF, _NCHUNK):
        compute_rows(j)
    c2.wait()

    lane = lax.iota(jnp.int32, _L)
    outs = []
    for j in range(_NCHUNK):
        slot = j % _NBUF
        copies[slot].wait()
        for i in range(_GPC):
            g = j * _CHUNK + i * _L
            s = pl.ds(g, _L)
            val_v[s] = plsc.load_gather(bufs.at[slot],
                                        [lane + i * _L, x2_v[s]])
        if j + _NBUF < _NCHUNK:
            copies[slot] = fire(j + _NBUF, slot)
        c = pl.ds(j * _CHUNK, _CHUNK)
        outs.append(pltpu.async_copy(
            val_v.at[c], out_hbm.at[pl.ds(base + j * _CHUNK, _CHUNK)],
            sem_out))
    for o in outs:
        o.wait()


@jax.jit
def _gather_sc(x0, x1, x2, board):
    mesh = plsc.VectorSubcoreMesh(core_axis_name="c", subcore_axis_name="s")
    f = pl.kernel(
        _gather_body,
        out_type=jax.ShapeDtypeStruct((_B,), jnp.float32),
        mesh=mesh,
        compiler_params=pltpu.CompilerParams(needs_layout_passes=False),
        scratch_types=[
            pltpu.VMEM((_BPW,), jnp.int32),
            pltpu.VMEM((_BPW,), jnp.int32),
            pltpu.VMEM((_BPW,), jnp.int32),
            pltpu.VMEM((_BPW,), jnp.float32),
            pltpu.VMEM((_NBUF, _CHUNK, _V), jnp.float32),
            pltpu.SemaphoreType.DMA,
            pltpu.SemaphoreType.DMA,
            pltpu.SemaphoreType.DMA((_NBUF,)),
        ],
    )
    return f(x0, x1, x2, board)


def kernel(x0, x1, x2, board):
    x0 = x0.astype(jnp.int32)
    x1 = x1.astype(jnp.int32)
    x2 = x2.astype(jnp.int32)
    out = _gather_sc(x0, x1, x2, board)
    return out[:, None]


# speedup vs baseline: 3.3293x; 1.2836x over previous
import jax
import jax.numpy as jnp
from jax import lax
from jax.experimental import pallas as pl
from jax.experimental.pallas import tpu as pltpu
from jax.experimental.pallas import tpu_sc as plsc

_B = 16384
_V = 256
_NC = 2
_NS = 16
_NW = _NC * _NS
_BPW = _B // _NW
_L = 16
_GRAN = 16
_NROWS = _V * _V * _V // _GRAN
_CHUNK = 128
_NCHUNK = _BPW // _CHUNK
_NBUF = 3
_GPC = _CHUNK // _L


def _gather_body(x0_hbm, x1_hbm, x2_hbm, board_hbm, out_hbm,
                 x0_v, x2_v, row_v, lane_v, val_v, bufs,
                 sem_in, sem_out, sems):
    wid = lax.axis_index("s") * _NC + lax.axis_index("c")
    base = wid * _BPW

    c0 = pltpu.async_copy(x0_hbm.at[pl.ds(base, _BPW)], x0_v, sem_in)
    c1 = pltpu.async_copy(x1_hbm.at[pl.ds(base, _BPW)], row_v, sem_in)
    c2 = pltpu.async_copy(x2_hbm.at[pl.ds(base, _BPW)], x2_v, sem_out)
    c0.wait()
    c1.wait()
    c2.wait()

    def compute_rows(j):
        for i in range(_GPC):
            s = pl.ds(j * _CHUNK + i * _L, _L)
            x1v = row_v[s]
            x2v = x2_v[s]
            p = ((x0_v[s] << 16) | ((x1v >> 3) << 11) | ((x2v >> 7) << 10)
                 | ((x1v & 7) << 7) | (x2v & 127))
            row_v[s] = p >> 4
            lane_v[s] = p & 15

    def fire(j, slot):
        c = pl.ds(j * _CHUNK, _CHUNK)
        return pltpu.async_copy(board_hbm.at[row_v.at[c]],
                                bufs.at[slot], sems.at[slot])

    copies = []
    for j in range(_NBUF):
        compute_rows(j)
        copies.append(fire(j, j))
    for j in range(_NBUF, _NCHUNK):
        compute_rows(j)

    lane = lax.iota(jnp.int32, _L)
    outs = []
    for j in range(_NCHUNK):
        slot = j % _NBUF
        copies[slot].wait()
        for i in range(_GPC):
            g = j * _CHUNK + i * _L
            s = pl.ds(g, _L)
            val_v[s] = plsc.load_gather(bufs.at[slot],
                                        [lane + i * _L, lane_v[s]])
        if j + _NBUF < _NCHUNK:
            copies[slot] = fire(j + _NBUF, slot)
        c = pl.ds(j * _CHUNK, _CHUNK)
        outs.append(pltpu.async_copy(
            val_v.at[c], out_hbm.at[pl.ds(base + j * _CHUNK, _CHUNK)],
            sem_out))
    for o in outs:
        o.wait()


@jax.jit
def _gather_sc(x0, x1, x2, board16):
    mesh = plsc.VectorSubcoreMesh(core_axis_name="c", subcore_axis_name="s")
    f = pl.kernel(
        _gather_body,
        out_type=jax.ShapeDtypeStruct((_B,), jnp.float32),
        mesh=mesh,
        compiler_params=pltpu.CompilerParams(needs_layout_passes=False, use_tc_tiling_on_sc=False),
        scratch_types=[
            pltpu.VMEM((_BPW,), jnp.int32),
            pltpu.VMEM((_BPW,), jnp.int32),
            pltpu.VMEM((_BPW,), jnp.int32),
            pltpu.VMEM((_BPW,), jnp.int32),
            pltpu.VMEM((_BPW,), jnp.float32),
            pltpu.VMEM((_NBUF, _CHUNK, _GRAN), jnp.float32),
            pltpu.SemaphoreType.DMA,
            pltpu.SemaphoreType.DMA,
            pltpu.SemaphoreType.DMA((_NBUF,)),
        ],
    )
    return f(x0, x1, x2, board16)


def kernel(x0, x1, x2, board):
    x0 = x0.astype(jnp.int32)
    x1 = x1.astype(jnp.int32)
    x2 = x2.astype(jnp.int32)
    board16 = (board.reshape(_V, 32, 8, 2, 128)
               .transpose(0, 1, 3, 2, 4)
               .reshape(_NROWS, _GRAN))
    out = _gather_sc(x0, x1, x2, board16)
    return out[:, None]


# speedup vs baseline: 3.3570x; 1.0083x over previous
import jax
import jax.numpy as jnp
from jax import lax
from jax.experimental import pallas as pl
from jax.experimental.pallas import tpu as pltpu
from jax.experimental.pallas import tpu_sc as plsc

_B = 16384
_V = 256
_NC = 2
_NS = 16
_NW = _NC * _NS
_BPW = _B // _NW
_L = 16
_GRAN = 16
_NROWS = _V * _V * _V // _GRAN
_CHUNK = 128
_NCHUNK = _BPW // _CHUNK
_NGRP = _BPW // _L


def _gather_body(x0_hbm, x1_hbm, x2_hbm, board_hbm, out_hbm,
                 ints, val_v, buf, sem_in, sem_g):
    wid = lax.axis_index("s") * _NC + lax.axis_index("c")
    base = wid * _BPW

    c0 = pltpu.async_copy(x0_hbm.at[pl.ds(base, _BPW)], ints.at[0], sem_in)
    c1 = pltpu.async_copy(x1_hbm.at[pl.ds(base, _BPW)], ints.at[1], sem_in)
    c2 = pltpu.async_copy(x2_hbm.at[pl.ds(base, _BPW)], ints.at[2], sem_in)
    c0.wait()
    c1.wait()
    c2.wait()

    for i in range(_NGRP):
        s = pl.ds(i * _L, _L)
        x1v = ints.at[1][s]
        x2v = ints.at[2][s]
        p = ((ints.at[0][s] << 16) | ((x1v >> 3) << 11) | ((x2v >> 7) << 10)
             | ((x1v & 7) << 7) | (x2v & 127))
        ints.at[1][s] = p >> 4
        ints.at[2][s] = p & 15

    copies = [
        pltpu.async_copy(board_hbm.at[ints.at[1].at[pl.ds(j * _CHUNK, _CHUNK)]],
                         buf.at[pl.ds(j * _CHUNK, _CHUNK)], sem_g)
        for j in range(_NCHUNK)
    ]
    for cp in copies:
        cp.wait()

    lane = lax.iota(jnp.int32, _L)
    for i in range(_NGRP):
        s = pl.ds(i * _L, _L)
        val_v[s] = plsc.load_gather(buf, [lane + i * _L, ints.at[2][s]])

    pltpu.sync_copy(val_v, out_hbm.at[pl.ds(base, _BPW)])


@jax.jit
def _gather_sc(x0, x1, x2, board16):
    mesh = plsc.VectorSubcoreMesh(core_axis_name="c", subcore_axis_name="s")
    f = pl.kernel(
        _gather_body,
        out_type=jax.ShapeDtypeStruct((_B,), jnp.float32),
        mesh=mesh,
        compiler_params=pltpu.CompilerParams(
            needs_layout_passes=False, use_tc_tiling_on_sc=False),
        scratch_types=[
            pltpu.VMEM((3, _BPW), jnp.int32),
            pltpu.VMEM((_BPW,), jnp.float32),
            pltpu.VMEM((_BPW, _GRAN), jnp.float32),
            pltpu.SemaphoreType.DMA,
            pltpu.SemaphoreType.DMA,
        ],
    )
    return f(x0, x1, x2, board16)


def kernel(x0, x1, x2, board):
    x0 = x0.astype(jnp.int32)
    x1 = x1.astype(jnp.int32)
    x2 = x2.astype(jnp.int32)
    board16 = (board.reshape(_V, 32, 8, 2, 128)
               .transpose(0, 1, 3, 2, 4)
               .reshape(_NROWS, _GRAN))
    out = _gather_sc(x0, x1, x2, board16)
    return out[:, None]


# speedup vs baseline: 3.3995x; 1.0127x over previous
import jax
import jax.numpy as jnp
from jax import lax
from jax.experimental import pallas as pl
from jax.experimental.pallas import tpu as pltpu
from jax.experimental.pallas import tpu_sc as plsc

_B = 16384
_V = 256
_NC = 2
_NS = 16
_NW = _NC * _NS
_BPW = _B // _NW
_L = 16
_GRAN = 16
_NROWS = _V * _V * _V // _GRAN
_CHUNK = 128
_NCHUNK = _BPW // _CHUNK
_NGRP = _BPW // _L


def _gather_body(x0_hbm, x1_hbm, x2_hbm, board_hbm, out_hbm,
                 ints, val_v, buf, sem_in, sems):
    wid = lax.axis_index("s") * _NC + lax.axis_index("c")
    base = wid * _BPW

    c0 = pltpu.async_copy(x0_hbm.at[pl.ds(base, _BPW)], ints.at[0], sem_in)
    c1 = pltpu.async_copy(x1_hbm.at[pl.ds(base, _BPW)], ints.at[1], sem_in)
    c2 = pltpu.async_copy(x2_hbm.at[pl.ds(base, _BPW)], ints.at[2], sem_in)
    c0.wait()
    c1.wait()
    c2.wait()

    gpc = _CHUNK // _L
    copies = []
    for j in range(_NCHUNK):
        for i in range(j * gpc, (j + 1) * gpc):
            s = pl.ds(i * _L, _L)
            x1v = ints.at[1][s]
            x2v = ints.at[2][s]
            p = ((ints.at[0][s] << 16) | ((x1v >> 3) << 11)
                 | ((x2v >> 7) << 10) | ((x1v & 7) << 7) | (x2v & 127))
            ints.at[1][s] = p >> 4
            ints.at[2][s] = p & 15
        c = pl.ds(j * _CHUNK, _CHUNK)
        copies.append(pltpu.async_copy(board_hbm.at[ints.at[1].at[c]],
                                       buf.at[c], sems.at[j]))

    lane = lax.iota(jnp.int32, _L)
    outs = []
    for j in range(_NCHUNK):
        copies[j].wait()
        for i in range(j * gpc, (j + 1) * gpc):
            s = pl.ds(i * _L, _L)
            val_v[s] = plsc.load_gather(buf, [lane + i * _L, ints.at[2][s]])
        c = pl.ds(j * _CHUNK, _CHUNK)
        outs.append(pltpu.async_copy(
            val_v.at[c], out_hbm.at[pl.ds(base + j * _CHUNK, _CHUNK)],
            sem_in))
    for o in outs:
        o.wait()


@jax.jit
def _gather_sc(x0, x1, x2, board16):
    mesh = plsc.VectorSubcoreMesh(core_axis_name="c", subcore_axis_name="s")
    f = pl.kernel(
        _gather_body,
        out_type=jax.ShapeDtypeStruct((_B,), jnp.float32),
        mesh=mesh,
        compiler_params=pltpu.CompilerParams(
            needs_layout_passes=False, use_tc_tiling_on_sc=False),
        scratch_types=[
            pltpu.VMEM((3, _BPW), jnp.int32),
            pltpu.VMEM((_BPW,), jnp.float32),
            pltpu.VMEM((_BPW, _GRAN), jnp.float32),
            pltpu.SemaphoreType.DMA,
            pltpu.SemaphoreType.DMA((_NCHUNK,)),
        ],
    )
    return f(x0, x1, x2, board16)


def kernel(x0, x1, x2, board):
    x0 = x0.astype(jnp.int32)
    x1 = x1.astype(jnp.int32)
    x2 = x2.astype(jnp.int32)
    board16 = (board.reshape(_V, 32, 8, 2, 128)
               .transpose(0, 1, 3, 2, 4)
               .reshape(_NROWS, _GRAN))
    out = _gather_sc(x0, x1, x2, board16)
    return out[:, None]
